# baseline (speedup 1.0000x reference)
import jax
import jax.numpy as jnp
from jax.experimental import pallas as pl
from jax.experimental.pallas import tpu as pltpu

_K = 20
_HEAD = 128
_RB = 256

_NOISE_CACHE = [None]


def _noise_table(shape, dtype):
    def build():
        return (jax.random.uniform(jax.random.key(42), shape, dtype)
                * 0.01).reshape(shape[0] * shape[1], shape[2], shape[3])

    if _NOISE_CACHE[0] is None or _NOISE_CACHE[0][0] != (shape, dtype):
        try:
            with jax.ensure_compile_time_eval():
                noise = build()
            if not isinstance(noise, jax.core.Tracer):
                _NOISE_CACHE[0] = ((shape, dtype), noise)
        except Exception:
            pass
    if _NOISE_CACHE[0] is not None and _NOISE_CACHE[0][0] == (shape, dtype):
        return _NOISE_CACHE[0][1]
    return build()


def _mask_pass_kernel(adj_ref, noise_ref, m_ref, acc_ref):
    rb = pl.program_id(1)
    nrb = pl.num_programs(1)
    a = adj_ref[0]
    y = jnp.abs(a) + noise_ref[0]
    rows, n = y.shape
    col = jax.lax.broadcasted_iota(jnp.int32, (rows, n), 1)

    pres_rows = []
    vals = y
    for _ in range(_K):
        mx = jnp.max(vals, axis=1, keepdims=True)
        am = jnp.min(jnp.where(vals == mx, col, n), axis=1,
                     keepdims=True)
        e = col == am
        vals = jnp.where(e, -jnp.inf, vals)
        pres = jnp.any(e, axis=0, keepdims=True)
        pres_rows.append(pres.astype(jnp.float32))

    mblk_t = jnp.concatenate(
        pres_rows + [jnp.zeros((32 - _K, n), jnp.float32)], axis=0)

    @pl.when(rb == 0)
    def _():
        acc_ref[...] = mblk_t

    @pl.when(rb != 0)
    def _():
        acc_ref[...] = jnp.maximum(acc_ref[...], mblk_t)

    @pl.when(rb == nrb - 1)
    def _():
        mt = acc_ref[...].T
        m_ref[0] = jnp.concatenate(
            [mt, jnp.zeros((n, _HEAD - 32), jnp.float32)], axis=1)


def _apply_kernel(adj_ref, m_ref, out_ref):
    a = adj_ref[0]
    out_ref[0] = a * jnp.float32(1e-7)
    head = a[:, :_HEAD] * (jnp.float32(1e-7) + m_ref[0])
    out_ref[0, :, 0:_HEAD] = head


def kernel(adj):
    L, B, N, _ = adj.shape
    LB = L * B
    a3 = adj.reshape(LB, N, N)
    noise = _noise_table(adj.shape, adj.dtype)
    rb = min(_RB, N)
    nrb = N // rb

    m = pl.pallas_call(
        _mask_pass_kernel,
        grid=(LB, nrb),
        in_specs=[
            pl.BlockSpec((1, rb, N), lambda i, j: (i, j, 0)),
            pl.BlockSpec((1, rb, N), lambda i, j: (i, j, 0)),
        ],
        out_specs=pl.BlockSpec((1, N, _HEAD), lambda i, j: (i, 0, 0)),
        out_shape=jax.ShapeDtypeStruct((LB, N, _HEAD), jnp.float32),
        scratch_shapes=[pltpu.VMEM((32, N), jnp.float32)],
    )(a3, noise)

    out = pl.pallas_call(
        _apply_kernel,
        grid=(LB, nrb),
        in_specs=[
            pl.BlockSpec((1, rb, N), lambda i, j: (i, j, 0)),
            pl.BlockSpec((1, rb, _HEAD), lambda i, j: (i, j, 0)),
        ],
        out_specs=pl.BlockSpec((1, rb, N), lambda i, j: (i, j, 0)),
        out_shape=jax.ShapeDtypeStruct((LB, N, N), jnp.float32),
    )(a3, m)
    return out.reshape(L, B, N, N)


# speedup vs baseline: 1.0584x; 1.0584x over previous
import jax
import jax.numpy as jnp
from jax.experimental import pallas as pl
from jax.experimental.pallas import tpu as pltpu

_K = 20
_HEAD = 128
_RB = 256

_NOISE_CACHE = [None]


def _noise_table(shape, dtype):
    def build():
        return (jax.random.uniform(jax.random.key(42), shape, dtype)
                * 0.01).reshape(shape[0] * shape[1], shape[2], shape[3])

    if _NOISE_CACHE[0] is None or _NOISE_CACHE[0][0] != (shape, dtype):
        try:
            with jax.ensure_compile_time_eval():
                noise = build()
            if not isinstance(noise, jax.core.Tracer):
                _NOISE_CACHE[0] = ((shape, dtype), noise)
        except Exception:
            pass
    if _NOISE_CACHE[0] is not None and _NOISE_CACHE[0][0] == (shape, dtype):
        return _NOISE_CACHE[0][1]
    return build()


def _fused_kernel(nslab, adj_m_ref, noise_ref, adj_a_ref, out_ref,
                  acc_a, acc_b, fin_a, fin_b):
    i = pl.program_id(0)
    j = pl.program_id(1)
    nrb = pl.num_programs(1)
    even = i % 2 == 0

    @pl.when(i < nslab)
    def _():
        a = adj_m_ref[0]
        y = jnp.abs(a) + noise_ref[0]
        rows, n = y.shape
        col = jax.lax.broadcasted_iota(jnp.int32, (rows, n), 1)

        pres_rows = []
        vals = y
        for _ in range(_K):
            mx = jnp.max(vals, axis=1, keepdims=True)
            am = jnp.min(jnp.where(vals == mx, col, n), axis=1,
                         keepdims=True)
            e = col == am
            vals = jnp.where(e, -jnp.inf, vals)
            pres = jnp.any(e, axis=0, keepdims=True)
            pres_rows.append(pres.astype(jnp.float32))

        mblk_t = jnp.concatenate(
            pres_rows + [jnp.zeros((32 - _K, n), jnp.float32)], axis=0)

        def accum(acc):
            @pl.when(j == 0)
            def _():
                acc[...] = mblk_t

            @pl.when(j != 0)
            def _():
                acc[...] = jnp.maximum(acc[...], mblk_t)

        @pl.when(even)
        def _():
            accum(acc_a)

        @pl.when(jnp.logical_not(even))
        def _():
            accum(acc_b)

        @pl.when(j == nrb - 1)
        def _():
            def finish(acc, fin):
                mt = acc[...].T
                fin[...] = jnp.concatenate(
                    [mt, jnp.zeros((n, _HEAD - 32), jnp.float32)], axis=1)

            @pl.when(even)
            def _():
                finish(acc_a, fin_a)

            @pl.when(jnp.logical_not(even))
            def _():
                finish(acc_b, fin_b)

    a2 = adj_a_ref[0]
    rb = a2.shape[0]
    m_prev = jnp.where(even, fin_b[pl.ds(j * rb, rb), :],
                       fin_a[pl.ds(j * rb, rb), :])
    out_ref[0] = a2 * jnp.float32(1e-7)
    out_ref[0, :, 0:_HEAD] = a2[:, :_HEAD] * (jnp.float32(1e-7) + m_prev)


def kernel(adj):
    import functools
    L, B, N, _ = adj.shape
    LB = L * B
    a3 = adj.reshape(LB, N, N)
    noise = _noise_table(adj.shape, adj.dtype)
    rb = min(_RB, N)
    nrb = N // rb
    last = LB - 1

    out = pl.pallas_call(
        functools.partial(_fused_kernel, LB),
        grid=(LB + 1, nrb),
        in_specs=[
            pl.BlockSpec((1, rb, N), lambda i, j: (jnp.minimum(i, last), j, 0)),
            pl.BlockSpec((1, rb, N), lambda i, j: (jnp.minimum(i, last), j, 0)),
            pl.BlockSpec((1, rb, N), lambda i, j: (jnp.maximum(i - 1, 0), j, 0)),
        ],
        out_specs=pl.BlockSpec((1, rb, N), lambda i, j: (jnp.maximum(i - 1, 0), j, 0)),
        out_shape=jax.ShapeDtypeStruct((LB, N, N), jnp.float32),
        scratch_shapes=[
            pltpu.VMEM((32, N), jnp.float32),
            pltpu.VMEM((32, N), jnp.float32),
            pltpu.VMEM((N, _HEAD), jnp.float32),
            pltpu.VMEM((N, _HEAD), jnp.float32),
        ],
    )(a3, noise, a3)
    return out.reshape(L, B, N, N)
